# baseline (speedup 1.0000x reference)
import jax
import jax.numpy as jnp
from jax.experimental import pallas as pl
from jax.experimental.pallas import tpu as pltpu

_BOX_W = 5.0
_CLS_W = 1.0
_ALPHA = 0.25
_THR = 0.3
_BLK = 4000
_BIG = 1 << 20


def _yolo_block_kernel(bp_ref, cp_ref, gb_ref, gl_ref, out_ref):
    bp = bp_ref[0, 0]
    x = cp_ref[0, 0].T
    g = gb_ref[0]
    lab = gl_ref[0]

    px0 = bp[0:1, :]
    py0 = bp[1:2, :]
    px1 = bp[2:3, :]
    py1 = bp[3:4, :]
    gx0 = g[:, 0:1]
    gy0 = g[:, 1:2]
    gx1 = g[:, 2:3]
    gy1 = g[:, 3:4]

    area1 = (px1 - px0) * (py1 - py0)
    area2 = (gx1 - gx0) * (gy1 - gy0)
    ltx = jnp.maximum(px0, gx0)
    lty = jnp.maximum(py0, gy0)
    rbx = jnp.minimum(px1, gx1)
    rby = jnp.minimum(py1, gy1)
    wx = jnp.maximum(rbx - ltx, 0.0)
    wy = jnp.maximum(rby - lty, 0.0)
    inter = wx * wy
    union = area1 + area2 - inter
    iou = inter / union
    cx0 = jnp.minimum(px0, gx0)
    cy0 = jnp.minimum(py0, gy0)
    cx1 = jnp.maximum(px1, gx1)
    cy1 = jnp.maximum(py1, gy1)
    wcx = jnp.maximum(cx1 - cx0, 0.0)
    wcy = jnp.maximum(cy1 - cy0, 0.0)
    areac = wcx * wcy
    giou = iou - (areac - union) / areac

    best = jnp.max(giou, axis=0, keepdims=True)
    pos = best > _THR

    jiota = jax.lax.broadcasted_iota(jnp.int32, lab.shape, 0)
    packed_const = jiota * 128 + lab
    pk = jnp.min(jnp.where(giou >= best, packed_const, _BIG),
                 axis=0, keepdims=True)
    matched = jnp.bitwise_and(pk, 127)

    box_sum = jnp.sum(jnp.where(pos, 1.0 - best, 0.0))
    npos = jnp.sum(pos.astype(jnp.float32))

    ciota = jax.lax.broadcasted_iota(jnp.int32, (x.shape[0], 1), 0)
    eq = ciota == matched
    q = jax.nn.sigmoid(-x)
    p = 1.0 - q
    ce0 = -jnp.log(q)
    v = (p * p) * ce0
    s0 = jnp.sum(v, axis=0, keepdims=True)
    xl = jnp.sum(jnp.where(eq, x, 0.0), axis=0, keepdims=True)

    pl_ = jax.nn.sigmoid(xl)
    spl = jnp.log1p(jnp.exp(-jnp.abs(xl)))
    rel = jnp.maximum(xl, 0.0)
    fl0l = (1.0 - _ALPHA) * (pl_ * pl_) * (rel + spl)
    ql = 1.0 - pl_
    fl1l = _ALPHA * (ql * ql) * (rel - xl + spl)
    row = (1.0 - _ALPHA) * s0 + (fl1l - fl0l)
    cls_sum = jnp.sum(jnp.where(pos, row, 0.0))

    lane = jax.lax.broadcasted_iota(jnp.int32, (1, 128), 1)
    out_ref[0] = (jnp.where(lane == 0, box_sum, 0.0)
                  + jnp.where(lane == 1, cls_sum, 0.0)
                  + jnp.where(lane == 2, npos, 0.0))


def kernel(box_preds, cls_preds, gt_boxes, gt_labels):
    B, N, _ = box_preds.shape
    C = cls_preds.shape[-1]
    G = gt_boxes.shape[1]
    blk = _BLK if N % _BLK == 0 else N
    nb = N // blk

    bpt = box_preds.reshape(B, nb, blk, 4).transpose(0, 1, 3, 2)
    cpt = cls_preds.reshape(B, nb, blk, C)
    gl3 = gt_labels.reshape(B, G, 1).astype(jnp.int32)

    out = pl.pallas_call(
        _yolo_block_kernel,
        grid=(B, nb),
        in_specs=[
            pl.BlockSpec((1, 1, 4, blk), lambda b, i: (b, i, 0, 0)),
            pl.BlockSpec((1, 1, blk, C), lambda b, i: (b, i, 0, 0)),
            pl.BlockSpec((1, G, 4), lambda b, i: (b, 0, 0)),
            pl.BlockSpec((1, G, 1), lambda b, i: (b, 0, 0)),
        ],
        out_specs=pl.BlockSpec((1, 1, 128), lambda b, i: (b * nb + i, 0, 0)),
        out_shape=jax.ShapeDtypeStruct((B * nb, 1, 128), jnp.float32),
        compiler_params=pltpu.CompilerParams(
            dimension_semantics=("parallel", "arbitrary")),
    )(bpt, cpt, gt_boxes, gl3)

    total_box = jnp.sum(out[:, 0, 0])
    total_cls = jnp.sum(out[:, 0, 1])
    num = jnp.sum(out[:, 0, 2])
    return (_BOX_W * total_box + _CLS_W * total_cls) / num


# speedup vs baseline: 1.6249x; 1.6249x over previous
import jax
import jax.numpy as jnp
from jax.experimental import pallas as pl
from jax.experimental.pallas import tpu as pltpu

_BOX_W = 5.0
_CLS_W = 1.0
_ALPHA = 0.25
_THR = 0.3
_BLK = 4000
_BIG = 1 << 20


def _yolo_block_kernel(bp_ref, cp_ref, gb_ref, gl_ref, out_ref):
    bp = bp_ref[0, 0]
    x = cp_ref[0, 0]
    g = gb_ref[0]
    lab = gl_ref[0]

    px0 = bp[0:1, :]
    py0 = bp[1:2, :]
    px1 = bp[2:3, :]
    py1 = bp[3:4, :]
    gx0 = g[:, 0:1]
    gy0 = g[:, 1:2]
    gx1 = g[:, 2:3]
    gy1 = g[:, 3:4]

    pw = px1 - px0
    ph = py1 - py0
    area1 = pw * ph
    gw = gx1 - gx0
    gh = gy1 - gy0
    area2 = gw * gh
    sa = area1 + area2
    wxr = jnp.minimum(px1, gx1) - jnp.maximum(px0, gx0)
    wyr = jnp.minimum(py1, gy1) - jnp.maximum(py0, gy0)
    inter = jnp.maximum(wxr, 0.0) * jnp.maximum(wyr, 0.0)
    union = sa - inter
    areac = ((pw + gw) - wxr) * ((ph + gh) - wyr)
    f = inter / union + union / areac

    best = jnp.max(f, axis=0, keepdims=True)
    pos = best > (_THR + 1.0)

    jiota = jax.lax.broadcasted_iota(jnp.int32, lab.shape, 0)
    packed_const = jiota * 128 + lab
    pk = jnp.min(jnp.where(f >= best, packed_const, _BIG),
                 axis=0, keepdims=True)
    matched = jnp.bitwise_and(pk, 127)

    box_sum = jnp.sum(jnp.where(pos, 2.0 - best, 0.0))
    npos = jnp.sum(pos.astype(jnp.float32))

    ciota = jax.lax.broadcasted_iota(jnp.int32, (x.shape[0], 1), 0)
    eq = ciota == matched
    q = jax.nn.sigmoid(-x)
    p = 1.0 - q
    vneg = (p * p) * jnp.log(q)
    xsel = jnp.where(eq, x, 0.0)
    ones8 = jnp.ones((8, x.shape[0]), dtype=jnp.float32)
    s0n = jnp.dot(ones8, vneg)[0:1]
    xl = jnp.dot(ones8, xsel)[0:1]

    pl_ = jax.nn.sigmoid(xl)
    spl = jnp.log1p(jnp.exp(-jnp.abs(xl)))
    rel = jnp.maximum(xl, 0.0)
    fl0l = (1.0 - _ALPHA) * (pl_ * pl_) * (rel + spl)
    ql = 1.0 - pl_
    fl1l = _ALPHA * (ql * ql) * (rel - xl + spl)
    row = (fl1l - fl0l) - (1.0 - _ALPHA) * s0n
    cls_sum = jnp.sum(jnp.where(pos, row, 0.0))

    lane = jax.lax.broadcasted_iota(jnp.int32, (1, 128), 1)
    out_ref[0] = (jnp.where(lane == 0, box_sum, 0.0)
                  + jnp.where(lane == 1, cls_sum, 0.0)
                  + jnp.where(lane == 2, npos, 0.0))


def kernel(box_preds, cls_preds, gt_boxes, gt_labels):
    B, N, _ = box_preds.shape
    C = cls_preds.shape[-1]
    G = gt_boxes.shape[1]
    blk = _BLK if N % _BLK == 0 else N
    nb = N // blk

    bpt = box_preds.reshape(B, nb, blk, 4).transpose(0, 1, 3, 2)
    cpt = cls_preds.reshape(B, nb, blk, C).transpose(0, 1, 3, 2)
    gl3 = gt_labels.reshape(B, G, 1).astype(jnp.int32)

    out = pl.pallas_call(
        _yolo_block_kernel,
        grid=(B, nb),
        in_specs=[
            pl.BlockSpec((1, 1, 4, blk), lambda b, i: (b, i, 0, 0)),
            pl.BlockSpec((1, 1, C, blk), lambda b, i: (b, i, 0, 0)),
            pl.BlockSpec((1, G, 4), lambda b, i: (b, 0, 0)),
            pl.BlockSpec((1, G, 1), lambda b, i: (b, 0, 0)),
        ],
        out_specs=pl.BlockSpec((1, 1, 128), lambda b, i: (b * nb + i, 0, 0)),
        out_shape=jax.ShapeDtypeStruct((B * nb, 1, 128), jnp.float32),
        compiler_params=pltpu.CompilerParams(
            dimension_semantics=("parallel", "arbitrary")),
    )(bpt, cpt, gt_boxes, gl3)

    total_box = jnp.sum(out[:, 0, 0])
    total_cls = jnp.sum(out[:, 0, 1])
    num = jnp.sum(out[:, 0, 2])
    return (_BOX_W * total_box + _CLS_W * total_cls) / num
